# baseline (speedup 1.0000x reference)
import functools

import jax
import jax.numpy as jnp
from jax import lax
from jax.experimental import pallas as pl
from jax.experimental.pallas import tpu as pltpu
from jax.experimental.pallas import tpu_sc as plsc

BATCH = 16384
DIM = 64

_info = plsc.get_sparse_core_info()
_NC, _NS = _info.num_cores, _info.num_subcores
_NW = _NC * _NS
_BPW = BATCH // _NW
_CHUNK = 128
_SUB = 256
_L = _info.num_lanes


def _gather_body(ent_emb, ent_tr, rel_emb, rel_tr, ih, it, ir,
                 oh, ot, orr, oht, ott, ort,
                 ihv, itv, irv, rows_a, rows_b, sem_a, sem_b,
                 wsem_a, wsem_b):
    wid = lax.axis_index("s") * _NC + lax.axis_index("c")
    base = wid * _BPW

    for iv, isrc in ((ihv, ih), (itv, it), (irv, ir)):
        pltpu.sync_copy(isrc.at[pl.ds(base, _BPW)], iv)
        for j in range(_BPW // _L):
            sl = pl.ds(j * _L, _L)
            iv[sl] = jax.lax.shift_right_logical(iv[sl], 1)

    tasks = []
    for iv, table, out in (
            (ihv, ent_emb, oh),
            (itv, ent_emb, ot),
            (ihv, ent_tr, oht),
            (itv, ent_tr, ott),
            (irv, rel_emb, orr),
            (irv, rel_tr, ort)):
        for s in range(_BPW // _SUB):
            tasks.append((iv, s * _SUB, table, out))

    bufs = [(rows_a, sem_a, wsem_a), (rows_b, sem_b, wsem_b)]
    pending_write = [None, None]
    prev = None

    def _finish(p):
        pdescs, prows, pout, poff, pwsem, pb = p
        for c in pdescs:
            c.wait()
        pending_write[pb] = pltpu.async_copy(
            prows, pout.at[pl.ds(base + poff, _SUB)], pwsem)

    for k, (iv, off, table, out) in enumerate(tasks):
        b = k % 2
        rows, gsem, wsem = bufs[b]
        if pending_write[b] is not None:
            pending_write[b].wait()
            pending_write[b] = None
        descs = []
        for j in range(_SUB // _CHUNK):
            descs.append(pltpu.async_copy(
                table.at[iv.at[pl.ds(off + j * _CHUNK, _CHUNK)]],
                rows.at[pl.ds(j * _CHUNK, _CHUNK)],
                gsem))
        if prev is not None:
            _finish(prev)
        prev = (descs, rows, out, off, wsem, b)
    _finish(prev)
    for w in pending_write:
        if w is not None:
            w.wait()


def _sc_gather(ent_emb, ent_tr, rel_emb, rel_tr, ih, it, ir):
    mesh = plsc.VectorSubcoreMesh(core_axis_name="c", subcore_axis_name="s")
    row_ty = jax.ShapeDtypeStruct((BATCH, 2 * DIM), jnp.float32)
    fn = pl.kernel(
        _gather_body,
        mesh=mesh,
        out_type=[row_ty] * 6,
        scratch_types=[
            pltpu.VMEM((_BPW,), jnp.int32),
            pltpu.VMEM((_BPW,), jnp.int32),
            pltpu.VMEM((_BPW,), jnp.int32),
            pltpu.VMEM((_SUB, 2 * DIM), jnp.float32),
            pltpu.VMEM((_SUB, 2 * DIM), jnp.float32),
            pltpu.SemaphoreType.DMA,
            pltpu.SemaphoreType.DMA,
            pltpu.SemaphoreType.DMA,
            pltpu.SemaphoreType.DMA,
        ],
    )
    return fn(ent_emb, ent_tr, rel_emb, rel_tr, ih, it, ir)


def _score_body(ih_ref, it_ref, ir_ref, gh_ref, gt_ref, gr_ref,
                ght_ref, gtt_ref, grt_ref, o_ref):
    def _half(g_ref, idx):
        g = g_ref[...]
        odd = jax.lax.bitwise_and(idx, 1)[:, None] == 1
        return jnp.where(odd, g[:, DIM:], g[:, :DIM])

    ih = ih_ref[...]
    it = it_ref[...]
    ir = ir_ref[...]
    h = _half(gh_ref, ih)
    t = _half(gt_ref, it)
    r = _half(gr_ref, ir)
    ht = _half(ght_ref, ih)
    tt = _half(gtt_ref, it)
    rt = _half(grt_ref, ir)

    def _l2(x):
        n = jnp.sqrt(jnp.sum(x * x, axis=-1, keepdims=True))
        return x / jnp.maximum(n, 1e-12)

    ph = _l2(h + jnp.sum(h * ht, axis=-1, keepdims=True) * rt)
    pt = _l2(t + jnp.sum(t * tt, axis=-1, keepdims=True) * rt)
    ph = _l2(ph)
    pt = _l2(pt)
    rn = _l2(r)
    o_ref[...] = jnp.sum(jnp.abs(ph + rn - pt), axis=-1)


def _tc_score(ih, it, ir, gh, gt, gr, ght, gtt, grt):
    blk = 2048
    grid = BATCH // blk
    row_spec = pl.BlockSpec((blk, 2 * DIM), lambda i: (i, 0))
    idx_spec = pl.BlockSpec((blk,), lambda i: (i,))
    return pl.pallas_call(
        _score_body,
        grid=(grid,),
        in_specs=[idx_spec] * 3 + [row_spec] * 6,
        out_specs=pl.BlockSpec((blk,), lambda i: (i,)),
        out_shape=jax.ShapeDtypeStruct((BATCH,), jnp.float32),
    )(ih, it, ir, gh, gt, gr, ght, gtt, grt)


def kernel(batch_h, batch_t, batch_r, ent_embeddings, rel_embeddings,
           ent_transfer, rel_transfer):
    ih = batch_h.astype(jnp.int32)
    it = batch_t.astype(jnp.int32)
    ir = batch_r.astype(jnp.int32)
    e2 = ent_embeddings.reshape(-1, 2 * DIM)
    et2 = ent_transfer.reshape(-1, 2 * DIM)
    r2 = rel_embeddings.reshape(-1, 2 * DIM)
    rt2 = rel_transfer.reshape(-1, 2 * DIM)
    gh, gt, gr, ght, gtt, grt = _sc_gather(e2, et2, r2, rt2, ih, it, ir)
    return _tc_score(ih, it, ir, gh, gt, gr, ght, gtt, grt)


# speedup vs baseline: 1.9038x; 1.9038x over previous
import jax
import jax.numpy as jnp
from jax import lax
from jax.experimental import pallas as pl
from jax.experimental.pallas import tpu as pltpu
from jax.experimental.pallas import tpu_sc as plsc

BATCH = 16384
DIM = 64
ENT = 1000000
DPAD = 1048576

_info = plsc.get_sparse_core_info()
_NC, _NS = _info.num_cores, _info.num_subcores
_NW = _NC * _NS
_BPW = BATCH // _NW
_CH = 16
_NCHK = _BPW // _CH


def _dots_body(e_ref, t_ref, o_ref):
    o_ref[...] = jnp.sum(e_ref[...] * t_ref[...], axis=0)


def _tc_dots(entT, enttrT):
    blk = 8192
    return pl.pallas_call(
        _dots_body,
        grid=(DPAD // blk,),
        in_specs=[pl.BlockSpec((DIM, blk),
                               lambda i: (0, jnp.minimum(i, ENT // 8192)))] * 2,
        out_specs=pl.BlockSpec((blk,), lambda i: (i,)),
        out_shape=jax.ShapeDtypeStruct((DPAD,), jnp.float32),
    )(entT, enttrT)


def _rsqrt_nr(ss):
    ss = jnp.maximum(ss, 1e-30)
    y = plsc.bitcast(ss, jnp.int32)
    y = 0x5F3759DF - jax.lax.shift_right_logical(y, 1)
    f = plsc.bitcast(y, jnp.float32)
    hss = 0.5 * ss
    for _ in range(3):
        f = f * (1.5 - hss * f * f)
    return jnp.minimum(f, 1e12)


def _sc_body(entE, relE, relT, dots2, ih, it, ir, out,
             ihv, itv, irv, ird8, ihd7, itd7,
             bufs_flat, phs, pts, score_v, sems_flat):
    wid = lax.axis_index("s") * _NC + lax.axis_index("c")
    base = wid * _BPW

    pltpu.sync_copy(ih.at[pl.ds(base, _BPW)], ihv)
    pltpu.sync_copy(it.at[pl.ds(base, _BPW)], itv)
    pltpu.sync_copy(ir.at[pl.ds(base, _BPW)], irv)

    def prep(i, _):
        sl = pl.ds(i * 16, 16)
        ird8[sl] = jax.lax.shift_right_logical(irv[sl], 3)
        ihd7[sl] = jax.lax.shift_right_logical(ihv[sl], 7)
        itd7[sl] = jax.lax.shift_right_logical(itv[sl], 7)
        return 0

    lax.fori_loop(0, _BPW // 16, prep, 0)

    jv = lax.iota(jnp.int32, 16)
    zf = jnp.zeros((16,), jnp.float32)
    zi = jnp.zeros((16,), jnp.int32)

    sets = []
    for b in range(2):
        sets.append(tuple(bufs_flat[b * 6:(b + 1) * 6])
                    + tuple(sems_flat[b * 2:(b + 1) * 2]))

    def _fire(c, s):
        bufH, bufT, bufR, bufRT, bufDH, bufDT, sem_w, sem_b = s
        off = c * _CH
        sl = pl.ds(off, _CH)
        pltpu.async_copy(relE.at[ird8.at[sl]], bufR, sem_b)
        pltpu.async_copy(relT.at[ird8.at[sl]], bufRT, sem_b)
        pltpu.async_copy(dots2.at[ihd7.at[sl]], bufDH, sem_b)
        pltpu.async_copy(dots2.at[itd7.at[sl]], bufDT, sem_b)
        ehv = ihv[sl]
        etv = itv[sl]
        for j in range(_CH):
            rbh = pl.multiple_of((ehv[j] >> 3) * 8, 8)
            rbt = pl.multiple_of((etv[j] >> 3) * 8, 8)
            pltpu.async_copy(entE.at[pl.ds(rbh, 8), :], bufH.at[j], sem_w)
            pltpu.async_copy(entE.at[pl.ds(rbt, 8), :], bufT.at[j], sem_w)

    def _drain(s):
        bufH, bufT, bufR, bufRT, bufDH, bufDT, sem_w, sem_b = s
        for j in range(_CH):
            pltpu.make_async_copy(entE.at[pl.ds(0, 8), :], bufH.at[j],
                                  sem_w).wait()
            pltpu.make_async_copy(entE.at[pl.ds(0, 8), :], bufT.at[j],
                                  sem_w).wait()
        pltpu.make_async_copy(relE.at[pl.ds(0, _CH)], bufR, sem_b).wait()
        pltpu.make_async_copy(relE.at[pl.ds(0, _CH)], bufRT, sem_b).wait()
        pltpu.make_async_copy(dots2.at[pl.ds(0, _CH)], bufDH, sem_b).wait()
        pltpu.make_async_copy(dots2.at[pl.ds(0, _CH)], bufDT, sem_b).wait()

    def _compute(c, s):
        bufH, bufT, bufR, bufRT, bufDH, bufDT, sem_w, sem_b = s
        off = c * _CH
        sl = pl.ds(off, _CH)
        ehv = ihv[sl]
        etv = itv[sl]
        erv = irv[sl]
        e8h = jax.lax.bitwise_and(ehv, 7)
        e8t = jax.lax.bitwise_and(etv, 7)
        orv = jax.lax.bitwise_and(erv, 7) * DIM
        dh = plsc.load_gather(bufDH, [jv, jax.lax.bitwise_and(ehv, 127)])
        dt = plsc.load_gather(bufDT, [jv, jax.lax.bitwise_and(etv, 127)])

        def p2(d, carry):
            ssh, sst, ssr = carry
            dv = zi + d
            h = plsc.load_gather(bufH, [jv, e8h, dv])
            t = plsc.load_gather(bufT, [jv, e8t, dv])
            r = plsc.load_gather(bufR, [jv, orv + d])
            rt = plsc.load_gather(bufRT, [jv, orv + d])
            ph = h + dh * rt
            pt = t + dt * rt
            phs[d] = ph
            pts[d] = pt
            return (ssh + ph * ph, sst + pt * pt, ssr + r * r)

        ssh, sst, ssr = lax.fori_loop(0, DIM, p2, (zf, zf, zf), unroll=8)

        invh = _rsqrt_nr(ssh)
        invt = _rsqrt_nr(sst)
        invr = _rsqrt_nr(ssr)

        def p4(d, sacc):
            r = plsc.load_gather(bufR, [jv, orv + d])
            return sacc + jnp.abs(phs[d] * invh + r * invr - pts[d] * invt)

        sacc = lax.fori_loop(0, DIM, p4, zf, unroll=8)
        score_v[sl] = sacc

    _fire(0, sets[0])
    _fire(1, sets[1])

    def outer(i, _):
        c0 = i * 2
        _drain(sets[0])
        _compute(c0, sets[0])

        @pl.when(c0 + 2 < _NCHK)
        def _():
            _fire(c0 + 2, sets[0])

        _drain(sets[1])
        _compute(c0 + 1, sets[1])

        @pl.when(c0 + 3 < _NCHK)
        def _():
            _fire(c0 + 3, sets[1])

        return 0

    lax.fori_loop(0, _NCHK // 2, outer, 0)

    pltpu.sync_copy(score_v, out.at[pl.ds(base, _BPW)])


def kernel(batch_h, batch_t, batch_r, ent_embeddings, rel_embeddings,
           ent_transfer, rel_transfer):
    ih = batch_h.astype(jnp.int32)
    it = batch_t.astype(jnp.int32)
    ir = batch_r.astype(jnp.int32)

    dots = _tc_dots(ent_embeddings.T, ent_transfer.T)
    dots2 = dots.reshape(DPAD // 128, 128)
    relE = rel_embeddings.reshape(125, 512)
    relT = rel_transfer.reshape(125, 512)

    mesh = plsc.VectorSubcoreMesh(core_axis_name="c", subcore_axis_name="s")
    buf_types = []
    sem_types = []
    for _ in range(2):
        buf_types += [
            pltpu.VMEM((_CH, 8, DIM), jnp.float32),
            pltpu.VMEM((_CH, 8, DIM), jnp.float32),
            pltpu.VMEM((_CH, 512), jnp.float32),
            pltpu.VMEM((_CH, 512), jnp.float32),
            pltpu.VMEM((_CH, 128), jnp.float32),
            pltpu.VMEM((_CH, 128), jnp.float32),
        ]
        sem_types += [pltpu.SemaphoreType.DMA, pltpu.SemaphoreType.DMA]

    fn = pl.kernel(
        lambda entE, relE_, relT_, dots2_, ih_, it_, ir_, out, *scr:
            _sc_body(entE, relE_, relT_, dots2_, ih_, it_, ir_, out,
                     scr[0], scr[1], scr[2], scr[3], scr[4], scr[5],
                     scr[6:18], scr[18], scr[19], scr[20], scr[21:25]),
        mesh=mesh,
        compiler_params=pltpu.CompilerParams(use_tc_tiling_on_sc=True,
                                             needs_layout_passes=False),
        out_type=jax.ShapeDtypeStruct((BATCH,), jnp.float32),
        scratch_types=(
            [pltpu.VMEM((_BPW,), jnp.int32)] * 6
            + buf_types
            + [pltpu.VMEM((DIM, 16), jnp.float32)] * 2
            + [pltpu.VMEM((_BPW,), jnp.float32)]
            + sem_types
        ),
    )
    return fn(ent_embeddings, relE, relT, dots2, ih, it, ir)


# speedup vs baseline: 2.5793x; 1.3548x over previous
import jax
import jax.numpy as jnp
from jax import lax
from jax.experimental import pallas as pl
from jax.experimental.pallas import tpu as pltpu
from jax.experimental.pallas import tpu_sc as plsc

BATCH = 16384
DIM = 64
ENT = 1000000
DPAD = 1048576

_info = plsc.get_sparse_core_info()
_NC, _NS = _info.num_cores, _info.num_subcores
_NW = _NC * _NS
_BPW = BATCH // _NW
_CH = 16
_NCHK = _BPW // _CH


def _prep_body(e_ref, t_ref, d_ref, o_ref):
    e = e_ref[...]
    d_ref[...] = jnp.sum(e * t_ref[...], axis=0)
    o_ref[...] = e.T


def _tc_prep(entT, enttrT):
    blk = 8192
    clamp = lambda i: (0, jnp.minimum(i, ENT // 8192))
    return pl.pallas_call(
        _prep_body,
        grid=(DPAD // blk,),
        in_specs=[pl.BlockSpec((DIM, blk), clamp)] * 2,
        out_specs=[
            pl.BlockSpec((blk,), lambda i: (i,)),
            pl.BlockSpec((blk, DIM), lambda i: (jnp.minimum(i, ENT // 8192), 0)),
        ],
        out_shape=[
            jax.ShapeDtypeStruct((DPAD,), jnp.float32),
            jax.ShapeDtypeStruct((ENT, DIM), jnp.float32),
        ],
    )(entT, enttrT)


def _rsqrt_nr(ss):
    ss = jnp.maximum(ss, 1e-30)
    y = plsc.bitcast(ss, jnp.int32)
    y = 0x5F3759DF - jax.lax.shift_right_logical(y, 1)
    f = plsc.bitcast(y, jnp.float32)
    hss = 0.5 * ss
    for _ in range(3):
        f = f * (1.5 - hss * f * f)
    return jnp.minimum(f, 1e12)


def _sc_body(entE, relE, relT, dots2, ih, it, ir, out,
             ihv, itv, irv, ird8, ihd7, itd7,
             bufs_flat, phs, pts, score_v, sems_flat):
    wid = lax.axis_index("s") * _NC + lax.axis_index("c")
    base = wid * _BPW

    pltpu.sync_copy(ih.at[pl.ds(base, _BPW)], ihv)
    pltpu.sync_copy(it.at[pl.ds(base, _BPW)], itv)
    pltpu.sync_copy(ir.at[pl.ds(base, _BPW)], irv)

    def prep(i, _):
        sl = pl.ds(i * 16, 16)
        ird8[sl] = jax.lax.shift_right_logical(irv[sl], 3)
        ihd7[sl] = jax.lax.shift_right_logical(ihv[sl], 7)
        itd7[sl] = jax.lax.shift_right_logical(itv[sl], 7)
        return 0

    lax.fori_loop(0, _BPW // 16, prep, 0)

    jv = lax.iota(jnp.int32, 16)
    zf = jnp.zeros((16,), jnp.float32)
    zi = jnp.zeros((16,), jnp.int32)

    sets = []
    for b in range(2):
        sets.append(tuple(bufs_flat[b * 6:(b + 1) * 6])
                    + tuple(sems_flat[b * 2:(b + 1) * 2]))

    def _fire(c, s):
        bufH, bufT, bufR, bufRT, bufDH, bufDT, sem_w, sem_b = s
        off = c * _CH
        sl = pl.ds(off, _CH)
        pltpu.async_copy(relE.at[ird8.at[sl]], bufR, sem_b)
        pltpu.async_copy(relT.at[ird8.at[sl]], bufRT, sem_b)
        pltpu.async_copy(dots2.at[ihd7.at[sl]], bufDH, sem_b)
        pltpu.async_copy(dots2.at[itd7.at[sl]], bufDT, sem_b)
        ehv = ihv[sl]
        etv = itv[sl]
        for j in range(_CH):
            rbh = pl.multiple_of((ehv[j] >> 3) * 8, 8)
            rbt = pl.multiple_of((etv[j] >> 3) * 8, 8)
            pltpu.async_copy(entE.at[pl.ds(rbh, 8), :], bufH.at[j], sem_w)
            pltpu.async_copy(entE.at[pl.ds(rbt, 8), :], bufT.at[j], sem_w)

    def _drain(s):
        bufH, bufT, bufR, bufRT, bufDH, bufDT, sem_w, sem_b = s
        for j in range(_CH):
            pltpu.make_async_copy(entE.at[pl.ds(0, 8), :], bufH.at[j],
                                  sem_w).wait()
            pltpu.make_async_copy(entE.at[pl.ds(0, 8), :], bufT.at[j],
                                  sem_w).wait()
        pltpu.make_async_copy(relE.at[pl.ds(0, _CH)], bufR, sem_b).wait()
        pltpu.make_async_copy(relE.at[pl.ds(0, _CH)], bufRT, sem_b).wait()
        pltpu.make_async_copy(dots2.at[pl.ds(0, _CH)], bufDH, sem_b).wait()
        pltpu.make_async_copy(dots2.at[pl.ds(0, _CH)], bufDT, sem_b).wait()

    def _compute(c, s):
        bufH, bufT, bufR, bufRT, bufDH, bufDT, sem_w, sem_b = s
        off = c * _CH
        sl = pl.ds(off, _CH)
        ehv = ihv[sl]
        etv = itv[sl]
        erv = irv[sl]
        e8h = jax.lax.bitwise_and(ehv, 7)
        e8t = jax.lax.bitwise_and(etv, 7)
        orv = jax.lax.bitwise_and(erv, 7) * DIM
        dh = plsc.load_gather(bufDH, [jv, jax.lax.bitwise_and(ehv, 127)])
        dt = plsc.load_gather(bufDT, [jv, jax.lax.bitwise_and(etv, 127)])

        def p2(d, carry):
            ssh, sst, ssr = carry
            dv = zi + d
            h = plsc.load_gather(bufH, [jv, e8h, dv])
            t = plsc.load_gather(bufT, [jv, e8t, dv])
            r = plsc.load_gather(bufR, [jv, orv + d])
            rt = plsc.load_gather(bufRT, [jv, orv + d])
            ph = h + dh * rt
            pt = t + dt * rt
            phs[d] = ph
            pts[d] = pt
            return (ssh + ph * ph, sst + pt * pt, ssr + r * r)

        ssh, sst, ssr = lax.fori_loop(0, DIM, p2, (zf, zf, zf), unroll=8)

        invh = _rsqrt_nr(ssh)
        invt = _rsqrt_nr(sst)
        invr = _rsqrt_nr(ssr)

        def p4(d, sacc):
            r = plsc.load_gather(bufR, [jv, orv + d])
            return sacc + jnp.abs(phs[d] * invh + r * invr - pts[d] * invt)

        sacc = lax.fori_loop(0, DIM, p4, zf, unroll=8)
        score_v[sl] = sacc

    _fire(0, sets[0])
    _fire(1, sets[1])

    def outer(i, _):
        c0 = i * 2
        _drain(sets[0])
        _compute(c0, sets[0])

        @pl.when(c0 + 2 < _NCHK)
        def _():
            _fire(c0 + 2, sets[0])

        _drain(sets[1])
        _compute(c0 + 1, sets[1])

        @pl.when(c0 + 3 < _NCHK)
        def _():
            _fire(c0 + 3, sets[1])

        return 0

    lax.fori_loop(0, _NCHK // 2, outer, 0)

    pltpu.sync_copy(score_v, out.at[pl.ds(base, _BPW)])


def kernel(batch_h, batch_t, batch_r, ent_embeddings, rel_embeddings,
           ent_transfer, rel_transfer):
    ih = batch_h.astype(jnp.int32)
    it = batch_t.astype(jnp.int32)
    ir = batch_r.astype(jnp.int32)

    dots, entE = _tc_prep(ent_embeddings.T, ent_transfer.T)
    dots2 = dots.reshape(DPAD // 128, 128)
    relE = rel_embeddings.reshape(125, 512)
    relT = rel_transfer.reshape(125, 512)

    mesh = plsc.VectorSubcoreMesh(core_axis_name="c", subcore_axis_name="s")
    buf_types = []
    sem_types = []
    for _ in range(2):
        buf_types += [
            pltpu.VMEM((_CH, 8, DIM), jnp.float32),
            pltpu.VMEM((_CH, 8, DIM), jnp.float32),
            pltpu.VMEM((_CH, 512), jnp.float32),
            pltpu.VMEM((_CH, 512), jnp.float32),
            pltpu.VMEM((_CH, 128), jnp.float32),
            pltpu.VMEM((_CH, 128), jnp.float32),
        ]
        sem_types += [pltpu.SemaphoreType.DMA, pltpu.SemaphoreType.DMA]

    fn = pl.kernel(
        lambda entE, relE_, relT_, dots2_, ih_, it_, ir_, out, *scr:
            _sc_body(entE, relE_, relT_, dots2_, ih_, it_, ir_, out,
                     scr[0], scr[1], scr[2], scr[3], scr[4], scr[5],
                     scr[6:18], scr[18], scr[19], scr[20], scr[21:25]),
        mesh=mesh,
        compiler_params=pltpu.CompilerParams(use_tc_tiling_on_sc=True,
                                             needs_layout_passes=False),
        out_type=jax.ShapeDtypeStruct((BATCH,), jnp.float32),
        scratch_types=(
            [pltpu.VMEM((_BPW,), jnp.int32)] * 6
            + buf_types
            + [pltpu.VMEM((DIM, 16), jnp.float32)] * 2
            + [pltpu.VMEM((_BPW,), jnp.float32)]
            + sem_types
        ),
    )
    return fn(entE, relE, relT, dots2, ih, it, ir)
